# baseline (speedup 1.0000x reference)
import functools

import jax
import jax.numpy as jnp
from jax import lax
from jax.experimental import pallas as pl
from jax.experimental.pallas import tpu as pltpu
from jax.experimental.pallas import tpu_sc as plsc

B = 64
S = 512
H = 768
V16 = H // 16
NC = 2
NS = 16
NW = NC * NS
SLAB = S // NW
TPW = B * SLAB
NB = 2
G = NB * SLAB
NCHUNK = TPW // G
NBUF = 4
EPS = 1e-05

_mesh = plsc.VectorSubcoreMesh(
    core_axis_name="c", subcore_axis_name="s", num_cores=NC, num_subcores=NS
)


def _hsum(v):
    lanes = lax.iota(jnp.int32, 16)
    for k in (1, 2, 4, 8):
        v = v + jnp.take_along_axis(v, lanes ^ k, axis=0,
                                    mode="promise_in_bounds")
    return v


def _body(ids_h, tti_h, word_h, comb_h, gam_h, bet_h, out_h,
          ids_v, tti_v, comb_v, gam_v, bet_v, rows, gsems, wsems):
    cid = lax.axis_index("c")
    sid = lax.axis_index("s")
    wid = sid * NC + cid

    pltpu.sync_copy(ids_h.at[wid], ids_v)
    pltpu.sync_copy(tti_h.at[wid], tti_v)
    pltpu.sync_copy(comb_h.at[pl.ds(wid * SLAB, SLAB)],
                    comb_v.at[pl.ds(0, SLAB)])
    pltpu.sync_copy(comb_h.at[pl.ds(S + wid * SLAB, SLAB)],
                    comb_v.at[pl.ds(SLAB, SLAB)])
    pltpu.sync_copy(gam_h, gam_v)
    pltpu.sync_copy(bet_h, bet_v)

    def start_gather(c, k):
        idx = ids_v.at[pl.ds(c * G, G)]
        pltpu.async_copy(word_h.at[idx], rows.at[k], gsems.at[k])

    def wait_gather(k):
        idx = ids_v.at[pl.ds(0, G)]
        pltpu.make_async_copy(word_h.at[idx], rows.at[k], gsems.at[k]).wait()

    def start_writes(c, k):
        for kb in range(NB):
            orow = (NB * c + kb) * S + wid * SLAB
            pltpu.async_copy(rows.at[k, pl.ds(kb * SLAB, SLAB)],
                             out_h.at[pl.ds(orow, SLAB)], wsems.at[k])

    def wait_writes(k):
        for kb in range(NB):
            pltpu.make_async_copy(
                rows.at[k, pl.ds(kb * SLAB, SLAB)],
                out_h.at[pl.ds(0, SLAB)], wsems.at[k]).wait()

    def ln_one(k, t, coff):
        s_loc = t & (SLAB - 1)
        tt16 = tti_v[pl.ds(coff + t - s_loc, 16)]
        lanes = lax.iota(jnp.int32, 16)
        selv = jnp.where(lanes == s_loc, tt16, 0)
        itt = _hsum(selv)[0]
        row = itt + s_loc
        acc_s = jnp.zeros((16,), jnp.float32)
        acc_q = jnp.zeros((16,), jnp.float32)
        for j in range(V16):
            sl = pl.ds(j * 16, 16)
            v = rows[k, t, sl] + comb_v[row, sl]
            rows[k, t, sl] = v
            acc_s = acc_s + v
            acc_q = acc_q + v * v
        mean = _hsum(acc_s)[0] * (1.0 / H)
        var = _hsum(acc_q)[0] * (1.0 / H) - mean * mean
        x = var + EPS
        i = lax.bitcast_convert_type(x, jnp.int32)
        i = jnp.int32(0x5F3759DF) - (i >> 1)
        ys = lax.bitcast_convert_type(i, jnp.float32)
        for _ in range(3):
            ys = ys * (1.5 - 0.5 * x * ys * ys)
        return mean, ys

    def process(c, k):
        coff = c * G

        @plsc.parallel_loop(0, G // 2)
        def pair(p):
            t0 = 2 * p
            t1 = 2 * p + 1
            m0, y0 = ln_one(k, t0, coff)
            m1, y1 = ln_one(k, t1, coff)
            for j in range(V16):
                sl = pl.ds(j * 16, 16)
                g = gam_v[sl]
                bb = bet_v[sl]
                v0 = rows[k, t0, sl]
                v1 = rows[k, t1, sl]
                rows[k, t0, sl] = (v0 - m0) * y0 * g + bb
                rows[k, t1, sl] = (v1 - m1) * y1 * g + bb
            return ()

    for k in range(NBUF):
        start_gather(k, k)

    def outer(pi, carry):
        c0 = NBUF * pi
        for k in range(NBUF):
            wait_gather(k)
            process(c0 + k, k)
            start_writes(c0 + k, k)
            if k >= 1:
                wait_writes(k - 1)

                @pl.when(c0 + NBUF + k - 1 < NCHUNK)
                def _():
                    start_gather(c0 + NBUF + k - 1, k - 1)

        wait_writes(NBUF - 1)

        @pl.when(c0 + 2 * NBUF - 1 < NCHUNK)
        def _():
            start_gather(c0 + 2 * NBUF - 1, NBUF - 1)

        return carry

    lax.fori_loop(0, NCHUNK // NBUF, outer, 0)


_emb_ln = functools.partial(
    pl.kernel,
    out_type=jax.ShapeDtypeStruct((B * S, H), jnp.float32),
    mesh=_mesh,
    scratch_types=[
        pltpu.VMEM((TPW,), jnp.int32),
        pltpu.VMEM((TPW,), jnp.int32),
        pltpu.VMEM((2 * SLAB, H), jnp.float32),
        pltpu.VMEM((H,), jnp.float32),
        pltpu.VMEM((H,), jnp.float32),
        pltpu.VMEM((NBUF, G, H), jnp.float32),
        pltpu.SemaphoreType.DMA((NBUF,)),
        pltpu.SemaphoreType.DMA((NBUF,)),
    ],
)(_body)


def kernel(input_ids, token_type_ids, word_embedding, segment_embedding,
           position_embedding, ln_gamma, ln_beta):
    ids = input_ids.astype(jnp.int32)
    ids_w = ids.reshape(B, NW, SLAB).transpose(1, 0, 2).reshape(NW, TPW)
    tti_w = (token_type_ids.astype(jnp.int32) * SLAB
             ).reshape(B, NW, SLAB).transpose(1, 0, 2).reshape(NW, TPW)
    comb = (position_embedding[None, :, :]
            + segment_embedding[:, None, :]).reshape(2 * S, H)
    out = _emb_ln(ids_w, tti_w, word_embedding, comb, ln_gamma, ln_beta)
    return out.reshape(B, S, H)


# speedup vs baseline: 2.1430x; 2.1430x over previous
import functools

import jax
import jax.numpy as jnp
from jax import lax
from jax.experimental import pallas as pl
from jax.experimental.pallas import tpu as pltpu
from jax.experimental.pallas import tpu_sc as plsc

B = 64
S = 512
H = 768
V16 = H // 16
NC = 2
NS = 16
NW = NC * NS
SLAB = S // NW
TPW = B * SLAB
NB = 2
G = NB * SLAB
NCHUNK = TPW // G
NBUF = 4
EPS = 1e-05

_mesh = plsc.VectorSubcoreMesh(
    core_axis_name="c", subcore_axis_name="s", num_cores=NC, num_subcores=NS
)


def _hsum(v):
    lanes = lax.iota(jnp.int32, 16)
    for k in (1, 2, 4, 8):
        v = v + jnp.take_along_axis(v, lanes ^ k, axis=0,
                                    mode="promise_in_bounds")
    return v


def _body(ids_h, tti_h, word_h, comb_h, gam_h, bet_h, out_h,
          ids_v, tti_v, comb_v, gam_v, bet_v, rows, gsems, wsems):
    cid = lax.axis_index("c")
    sid = lax.axis_index("s")
    wid = sid * NC + cid

    pltpu.sync_copy(ids_h.at[wid], ids_v)
    pltpu.sync_copy(tti_h.at[wid], tti_v)
    pltpu.sync_copy(comb_h.at[pl.ds(wid * SLAB, SLAB)],
                    comb_v.at[pl.ds(0, SLAB)])
    pltpu.sync_copy(comb_h.at[pl.ds(S + wid * SLAB, SLAB)],
                    comb_v.at[pl.ds(SLAB, SLAB)])
    pltpu.sync_copy(gam_h, gam_v)
    pltpu.sync_copy(bet_h, bet_v)

    def start_gather(c, k):
        idx = ids_v.at[pl.ds(c * G, G)]
        pltpu.async_copy(word_h.at[idx], rows.at[k], gsems.at[k])

    def wait_gather(k):
        idx = ids_v.at[pl.ds(0, G)]
        pltpu.make_async_copy(word_h.at[idx], rows.at[k], gsems.at[k]).wait()

    def start_writes(c, k):
        for kb in range(NB):
            orow = (NB * c + kb) * S + wid * SLAB
            pltpu.async_copy(rows.at[k, pl.ds(kb * SLAB, SLAB)],
                             out_h.at[pl.ds(orow, SLAB)], wsems.at[k])

    def wait_writes(k):
        for kb in range(NB):
            pltpu.make_async_copy(
                rows.at[k, pl.ds(kb * SLAB, SLAB)],
                out_h.at[pl.ds(0, SLAB)], wsems.at[k]).wait()

    def ln_one(k, t, coff):
        s_loc = t & (SLAB - 1)
        tt16 = tti_v[pl.ds(coff + t - s_loc, 16)]
        lanes = lax.iota(jnp.int32, 16)
        selv = jnp.where(lanes == s_loc, tt16, 0)
        itt = _hsum(selv)[0]
        row = itt + s_loc
        acc_s = jnp.zeros((16,), jnp.float32)
        acc_q = jnp.zeros((16,), jnp.float32)
        for j in range(V16):
            sl = pl.ds(j * 16, 16)
            v = rows[k, t, sl] + comb_v[row, sl]
            rows[k, t, sl] = v
            acc_s = acc_s + v
            acc_q = acc_q + v * v
        mean = _hsum(acc_s)[0] * (1.0 / H)
        var = _hsum(acc_q)[0] * (1.0 / H) - mean * mean
        x = var + EPS
        i = lax.bitcast_convert_type(x, jnp.int32)
        i = jnp.int32(0x5F3759DF) - (i >> 1)
        ys = lax.bitcast_convert_type(i, jnp.float32)
        for _ in range(3):
            ys = ys * (1.5 - 0.5 * x * ys * ys)
        return mean, ys

    def process(c, k):
        coff = c * G

        @plsc.parallel_loop(0, G)
        def tok(t):
            m0, y0 = ln_one(k, t, coff)
            for j in range(V16):
                sl = pl.ds(j * 16, 16)
                v0 = rows[k, t, sl]
                rows[k, t, sl] = (v0 - m0) * y0 * gam_v[sl] + bet_v[sl]
            return ()

    for k in range(NBUF):
        start_gather(k, k)

    def outer(c, carry):
        k = c & (NBUF - 1)
        wait_gather(k)
        process(c, k)
        start_writes(c, k)
        kp = (c - 1) & (NBUF - 1)

        @pl.when(c >= 1)
        def _():
            wait_writes(kp)

        @pl.when(jnp.logical_and(c >= 1, c + NBUF - 1 < NCHUNK))
        def _():
            start_gather(c + NBUF - 1, kp)

        return carry

    lax.fori_loop(0, NCHUNK, outer, 0)
    wait_writes((NCHUNK - 1) & (NBUF - 1))


_emb_ln = functools.partial(
    pl.kernel,
    out_type=jax.ShapeDtypeStruct((B * S, H), jnp.float32),
    mesh=_mesh,
    scratch_types=[
        pltpu.VMEM((TPW,), jnp.int32),
        pltpu.VMEM((TPW,), jnp.int32),
        pltpu.VMEM((2 * SLAB, H), jnp.float32),
        pltpu.VMEM((H,), jnp.float32),
        pltpu.VMEM((H,), jnp.float32),
        pltpu.VMEM((NBUF, G, H), jnp.float32),
        pltpu.SemaphoreType.DMA((NBUF,)),
        pltpu.SemaphoreType.DMA((NBUF,)),
    ],
)(_body)


def kernel(input_ids, token_type_ids, word_embedding, segment_embedding,
           position_embedding, ln_gamma, ln_beta):
    ids = input_ids.astype(jnp.int32)
    ids_w = ids.reshape(B, NW, SLAB).transpose(1, 0, 2).reshape(NW, TPW)
    tti_w = (token_type_ids.astype(jnp.int32) * SLAB
             ).reshape(B, NW, SLAB).transpose(1, 0, 2).reshape(NW, TPW)
    comb = (position_embedding[None, :, :]
            + segment_embedding[:, None, :]).reshape(2 * S, H)
    out = _emb_ln(ids_w, tti_w, word_embedding, comb, ln_gamma, ln_beta)
    return out.reshape(B, S, H)
